# Initial kernel scaffold; baseline (speedup 1.0000x reference)
import jax
import jax.numpy as jnp
from jax.experimental import pallas as pl


def kernel(logit, target):
    raise NotImplementedError("write your pallas kernel here")


# speedup vs baseline: 174.7830x; 174.7830x over previous
import jax
import jax.numpy as jnp
from jax.experimental import pallas as pl
from jax.experimental.pallas import tpu as pltpu

_GAMMA = 2.0
_SMOOTH = 1e-5


def _body(lg_ref, tg_ref, out_ref):
    C = lg_ref.shape[1]
    lg = lg_ref[0]
    tg = tg_ref[0, 0]
    total = jnp.sum(lg, axis=0)
    lg_t = jnp.zeros_like(total)
    for c in range(C):
        lg_t = jnp.where(tg == c, lg[c], lg_t)
    a = 1.0 - _SMOOTH - _SMOOTH / (C - 1)
    b = _SMOOTH / (C - 1)
    pt = a * lg_t + b * total + _SMOOTH
    one_m = 1.0 - pt
    loss = one_m * one_m * jnp.log(pt)
    i = pl.program_id(0)
    j = pl.program_id(1)
    out_ref[i, j] = -jnp.sum(loss)


def kernel(logit, target):
    B, C, H, W = logit.shape
    tgt = target.astype(jnp.int32)
    HB = 64
    grid = (B, H // HB)
    partials = pl.pallas_call(
        _body,
        grid=grid,
        in_specs=[
            pl.BlockSpec((1, C, HB, W), lambda i, j: (i, 0, j, 0)),
            pl.BlockSpec((1, 1, HB, W), lambda i, j: (i, 0, j, 0)),
        ],
        out_specs=pl.BlockSpec(memory_space=pltpu.SMEM),
        out_shape=jax.ShapeDtypeStruct(grid, jnp.float32),
    )(logit, tgt)
    return jnp.sum(partials) / (B * H * W)
